# baseline (speedup 1.0000x reference)
import functools

import jax
import jax.numpy as jnp
import numpy as np
from jax import lax
from jax.experimental import pallas as pl
from jax.experimental.pallas import tpu as pltpu
from jax.experimental.pallas import tpu_sc as plsc

F32 = jnp.float32
_LANES = 16
_CHUNK = 128
_NCORES = 2
_NSUB = 16
_NW = _NCORES * _NSUB


def _splat_lane(vec16, lane):
    idx = jnp.full((_LANES, 1), lane, jnp.int32)
    return lax.gather(
        vec16,
        idx,
        dimension_numbers=lax.GatherDimensionNumbers(
            offset_dims=(), collapsed_slice_dims=(0,), start_index_map=(0,)),
        slice_sizes=(1,),
        mode=lax.GatherScatterMode.PROMISE_IN_BOUNDS)


@functools.partial(jax.jit, static_argnums=(5, 6))
def _spmm(table, src_t, dst_t, w_t, zeros, n_nodes, nch):
    feat = 64
    nring = 4
    nsring = 2
    rpt = (-(-n_nodes // _NSUB) + 7) // 8 * 8
    rpt_last = n_nodes - (_NSUB - 1) * rpt
    assert rpt_last > 0 and rpt_last % 8 == 0
    assert nch % nring == 0
    mesh = plsc.VectorSubcoreMesh(core_axis_name="c", subcore_axis_name="s")

    def body(table_h, src_h, dst_h, w_h, zeros_h, out_h,
             src_v, dst_v, w_v, rows, sbuf, tbl, acc, gsems, ssems):
        c = lax.axis_index("c")
        s = lax.axis_index("s")
        wid = c * _NSUB + s
        pltpu.sync_copy(src_h.at[wid], src_v)
        pltpu.sync_copy(dst_h.at[wid], dst_v)
        pltpu.sync_copy(w_h.at[wid], w_v)
        r0 = s * rpt

        @pl.when(s < _NSUB - 1)
        def _():
            pltpu.sync_copy(zeros_h.at[pl.ds(r0, rpt)],
                            acc.at[pl.ds(r0, rpt)])
            pltpu.sync_copy(table_h.at[pl.ds(r0, rpt)],
                            tbl.at[pl.ds(r0, rpt)])

        @pl.when(s == _NSUB - 1)
        def _():
            pltpu.sync_copy(zeros_h.at[pl.ds(r0, rpt_last)],
                            acc.at[pl.ds(r0, rpt_last)])
            pltpu.sync_copy(table_h.at[pl.ds(r0, rpt_last)],
                            tbl.at[pl.ds(r0, rpt_last)])

        plsc.subcore_barrier()

        mask_hi = jnp.int32(-65536)

        def scale(rows_ref, sbuf_ref, j):
            for g in range(_CHUNK // _LANES):
                wv = w_v[j, pl.ds(g * _LANES, _LANES)]
                for l in range(_LANES):
                    ws = _splat_lane(wv, l)
                    e = g * _LANES + l
                    for q in range(2):
                        v = rows_ref[e, pl.ds(q * 32, 32)]
                        wrd = plsc.bitcast(v, jnp.int32)
                        lo = plsc.bitcast(lax.shift_left(wrd, 16), F32)
                        hi = plsc.bitcast(lax.bitwise_and(wrd, mask_hi), F32)
                        sbuf_ref[e, pl.ds(q * 32, _LANES)] = lo * ws
                        sbuf_ref[e, pl.ds(q * 32 + 16, _LANES)] = hi * ws

        for b in range(nring):
            pltpu.async_copy(tbl.at[src_v.at[b]], rows.at[b], gsems.at[b])

        def step(j, carry):
            b = lax.bitwise_and(j, nring - 1)
            sb = lax.bitwise_and(j, nsring - 1)
            pltpu.make_async_copy(
                tbl.at[src_v.at[j]], rows.at[b], gsems.at[b]).wait()

            @pl.when(j >= nsring)
            def _():
                pltpu.make_async_copy(
                    sbuf.at[sb], acc.at[dst_v.at[j]], ssems.at[sb]).wait()

            scale(rows.at[b], sbuf.at[sb], j)

            @pl.when(j + nring < nch)
            def _():
                pltpu.async_copy(tbl.at[src_v.at[j + nring]],
                                 rows.at[b], gsems.at[b])

            pltpu.async_copy(sbuf.at[sb], acc.at[dst_v.at[j]],
                             ssems.at[sb], add=True)
            return carry

        lax.fori_loop(0, nch, step, 0)
        for b in range(nsring):
            pltpu.make_async_copy(
                sbuf.at[b], acc.at[dst_v.at[nch - nsring + b]],
                ssems.at[b]).wait()
        plsc.subcore_barrier()

        @pl.when(s < _NSUB - 1)
        def _():
            pltpu.sync_copy(acc.at[pl.ds(r0, rpt)],
                            out_h.at[c, pl.ds(r0, rpt)])

        @pl.when(s == _NSUB - 1)
        def _():
            pltpu.sync_copy(acc.at[pl.ds(r0, rpt_last)],
                            out_h.at[c, pl.ds(r0, rpt_last)])

    call = pl.kernel(
        body,
        out_type=jax.ShapeDtypeStruct((_NCORES, n_nodes, feat), F32),
        mesh=mesh,
        scratch_types=[
            pltpu.VMEM((nch, _CHUNK), jnp.int32),
            pltpu.VMEM((nch, _CHUNK), jnp.int32),
            pltpu.VMEM((nch, _CHUNK), F32),
            pltpu.VMEM((nring, _CHUNK, feat), jnp.bfloat16),
            pltpu.VMEM((nsring, _CHUNK, feat), F32),
            pltpu.VMEM_SHARED((n_nodes, feat), jnp.bfloat16),
            pltpu.VMEM_SHARED((n_nodes, feat), F32),
            pltpu.SemaphoreType.DMA((nring,)),
            pltpu.SemaphoreType.DMA((nsring,)),
        ],
        compiler_params=pltpu.CompilerParams(use_tc_tiling_on_sc=False,
                                             needs_layout_passes=False),
    )
    return call(table, src_t, dst_t, w_t, zeros)


def _proj(x2d, w, block_rows=2000):
    rows, _ = x2d.shape
    din, dout = w.shape

    def body(x_ref, w_ref, o_ref):
        o_ref[...] = jnp.dot(
            x_ref[...], w_ref[...],
            preferred_element_type=F32).astype(jnp.bfloat16)

    return pl.pallas_call(
        body,
        grid=(rows // block_rows,),
        in_specs=[
            pl.BlockSpec((block_rows, din), lambda i: (i, 0)),
            pl.BlockSpec((din, dout), lambda i: (0, 0)),
        ],
        out_specs=pl.BlockSpec((block_rows, dout), lambda i: (i, 0)),
        out_shape=jax.ShapeDtypeStruct((rows, dout), jnp.bfloat16),
    )(x2d, w)


def _gru(p, h, bg, wr, wz, wn, ur, uz, un, brc, bzc, bin_, bhn, w_next=None,
         block_rows=2000):
    n, d = h.shape
    with_next = w_next is not None

    def body(*refs):
        if with_next:
            (p_ref, h_ref, bg_r, wr_r, wz_r, wn_r, ur_r, uz_r, un_r,
             brc_r, bzc_r, bin_r, bhn_r, wnx_r, ho_r, hw_r) = refs
        else:
            (p_ref, h_ref, bg_r, wr_r, wz_r, wn_r, ur_r, uz_r, un_r,
             brc_r, bzc_r, bin_r, bhn_r, ho_r) = refs
        dot = lambda a, m: jnp.dot(a, m[...], preferred_element_type=F32)
        xg = jnp.maximum(p_ref[0] + p_ref[1] + bg_r[...], 0.0)
        hh = h_ref[...]
        r = jax.nn.sigmoid(dot(xg, wr_r) + dot(hh, ur_r) + brc_r[...])
        z = jax.nn.sigmoid(dot(xg, wz_r) + dot(hh, uz_r) + bzc_r[...])
        nn = jnp.tanh(dot(xg, wn_r) + bin_r[...] + r * (dot(hh, un_r) + bhn_r[...]))
        h_new = (1.0 - z) * nn + z * hh
        ho_r[...] = h_new
        if with_next:
            hw_r[...] = dot(h_new, wnx_r).astype(jnp.bfloat16)

    mat = lambda: pl.BlockSpec((d, d), lambda i: (0, 0))
    vec = lambda: pl.BlockSpec((1, d), lambda i: (0, 0))
    in_specs = [
        pl.BlockSpec((2, block_rows, d), lambda i: (0, i, 0)),
        pl.BlockSpec((block_rows, d), lambda i: (i, 0)),
        vec(), mat(), mat(), mat(), mat(), mat(), mat(),
        vec(), vec(), vec(), vec(),
    ]
    args = [p, h, bg, wr, wz, wn, ur, uz, un, brc, bzc, bin_, bhn]
    out_spec = pl.BlockSpec((block_rows, d), lambda i: (i, 0))
    out_shape = jax.ShapeDtypeStruct((n, d), F32)
    if with_next:
        in_specs.append(mat())
        args.append(w_next)
        out_specs = [out_spec, out_spec]
        out_shapes = [out_shape, jax.ShapeDtypeStruct((n, d), jnp.bfloat16)]
    else:
        out_specs = out_spec
        out_shapes = out_shape
    return pl.pallas_call(
        body,
        grid=(n // block_rows,),
        in_specs=in_specs,
        out_specs=out_specs,
        out_shape=out_shapes,
    )(*args)


def _head(h1, ow1, ob1, ow2, ob2, block_rows=2000):
    n, d = h1.shape
    dhid = ow1.shape[1]
    steps = n // block_rows

    def body(h_ref, w1_ref, b1_ref, w2_ref, b2_ref, o_ref, acc_ref):
        i = pl.program_id(0)

        @pl.when(i == 0)
        def _():
            acc_ref[...] = jnp.zeros_like(acc_ref)

        acc_ref[...] += jnp.sum(h_ref[...], axis=0, keepdims=True)

        @pl.when(i == steps - 1)
        def _():
            g = acc_ref[...] * (1.0 / n)
            hid = jnp.maximum(
                jnp.dot(g, w1_ref[...], preferred_element_type=F32)
                + b1_ref[...], 0.0)
            o_ref[...] = jax.nn.sigmoid(
                jnp.dot(hid, w2_ref[...], preferred_element_type=F32)
                + b2_ref[...])

    return pl.pallas_call(
        body,
        grid=(steps,),
        in_specs=[
            pl.BlockSpec((block_rows, d), lambda i: (i, 0)),
            pl.BlockSpec((d, dhid), lambda i: (0, 0)),
            pl.BlockSpec((1, dhid), lambda i: (0, 0)),
            pl.BlockSpec((dhid, 1), lambda i: (0, 0)),
            pl.BlockSpec((1, 1), lambda i: (0, 0)),
        ],
        out_specs=pl.BlockSpec((1, 1), lambda i: (0, 0)),
        out_shape=jax.ShapeDtypeStruct((1, 1), F32),
        scratch_shapes=[pltpu.VMEM((1, d), F32)],
    )(h1, ow1, ob1.reshape(1, dhid), ow2, ob2.reshape(1, 1))


def _split_gru_params(wih, whh, bih, bhh, d):
    wr, wz, wn = wih[:, :d], wih[:, d:2 * d], wih[:, 2 * d:]
    ur, uz, un = whh[:, :d], whh[:, d:2 * d], whh[:, 2 * d:]
    brc = (bih[:d] + bhh[:d]).reshape(1, d)
    bzc = (bih[d:2 * d] + bhh[d:2 * d]).reshape(1, d)
    bin_ = bih[2 * d:].reshape(1, d)
    bhn = bhh[2 * d:].reshape(1, d)
    return wr, wz, wn, ur, uz, un, brc, bzc, bin_, bhn


def kernel(x_seq, edge_index, edge_weight, W0, b0, Wih0, Whh0, bih0, bhh0,
           W1, b1, Wih1, Whh1, bih1, bhh1, outW1, outb1, outW2, outb2):
    T, N, din = x_seq.shape
    dh = W0.shape[1]
    dout = W1.shape[1]
    E = edge_weight.shape[0]
    assert dh == 64 and dout == 64, "SC path specialized to 64-wide features"
    assert N % _NSUB == 0

    ept = -(-E // _NW)
    nch = -(-ept // _CHUNK)
    nch = -(-nch // 4) * 4
    pad_e = _NW * nch * _CHUNK - E
    src_t = jnp.pad(edge_index[0], (0, pad_e)).reshape(_NW, nch, _CHUNK)
    dst_t = jnp.pad(edge_index[1], (0, pad_e)).reshape(_NW, nch, _CHUNK)
    w_t = jnp.pad(edge_weight, (0, pad_e)).reshape(_NW, nch, _CHUNK)
    zeros = jnp.zeros((N, dh), F32)

    g0 = _split_gru_params(Wih0, Whh0, bih0, bhh0, dh)
    g1 = _split_gru_params(Wih1, Whh1, bih1, bhh1, dout)
    b0r = b0.reshape(1, dh)
    b1r = b1.reshape(1, dout)

    q = np.arange(dh) // 32
    r = np.arange(dh) % 32
    sigma = q * 32 + np.where(r < 16, 2 * r, 2 * (r - 16) + 1)
    tau = np.argsort(sigma)
    W0p = W0[:, tau]
    W1p = W1[:, tau]

    xw = _proj(x_seq.reshape(T * N, din), W0p).reshape(T, N, dh)

    h0 = jnp.zeros((N, dh), F32)
    h1 = jnp.zeros((N, dout), F32)
    p0s = [_spmm(xw[t], src_t, dst_t, w_t, zeros, N, nch) for t in range(T)]
    for t in range(T):
        p0 = p0s[t]
        h0, hw = _gru(p0, h0, b0r, *g0, w_next=W1p)
        p1 = _spmm(hw, src_t, dst_t, w_t, zeros, N, nch)
        h1 = _gru(p1, h1, b1r, *g1)
    return _head(h1, outW1, outb1, outW2, outb2)


# speedup vs baseline: 1.0863x; 1.0863x over previous
import functools

import jax
import jax.numpy as jnp
import numpy as np
from jax import lax
from jax.experimental import pallas as pl
from jax.experimental.pallas import tpu as pltpu
from jax.experimental.pallas import tpu_sc as plsc

F32 = jnp.float32
_LANES = 16
_CHUNK = 128
_NCORES = 2
_NSUB = 16
_NW = _NCORES * _NSUB


def _splat_lane(vec16, lane):
    idx = jnp.full((_LANES, 1), lane, jnp.int32)
    return lax.gather(
        vec16,
        idx,
        dimension_numbers=lax.GatherDimensionNumbers(
            offset_dims=(), collapsed_slice_dims=(0,), start_index_map=(0,)),
        slice_sizes=(1,),
        mode=lax.GatherScatterMode.PROMISE_IN_BOUNDS)


@functools.partial(jax.jit, static_argnums=(5, 6))
def _spmm(table, src_t, dst_t, w_t, zeros, n_nodes, nch):
    feat = 64
    nring = 4
    nsring = 2
    rpt = (-(-n_nodes // _NSUB) + 7) // 8 * 8
    rpt_last = n_nodes - (_NSUB - 1) * rpt
    assert rpt_last > 0 and rpt_last % 8 == 0
    assert nch % nring == 0
    mesh = plsc.VectorSubcoreMesh(core_axis_name="c", subcore_axis_name="s")

    def body(table_h, src_h, dst_h, w_h, zeros_h, out_h,
             src_v, dst_v, w_v, rows, sbuf, tbl, acc, gsems, ssems):
        c = lax.axis_index("c")
        s = lax.axis_index("s")
        wid = c * _NSUB + s
        pltpu.sync_copy(src_h.at[wid], src_v)
        pltpu.sync_copy(dst_h.at[wid], dst_v)
        pltpu.sync_copy(w_h.at[wid], w_v)
        r0 = s * rpt

        @pl.when(s < _NSUB - 1)
        def _():
            pltpu.sync_copy(zeros_h.at[pl.ds(r0, rpt)],
                            acc.at[pl.ds(r0, rpt)])
            pltpu.sync_copy(table_h.at[pl.ds(r0, rpt)],
                            tbl.at[pl.ds(r0, rpt)])

        @pl.when(s == _NSUB - 1)
        def _():
            pltpu.sync_copy(zeros_h.at[pl.ds(r0, rpt_last)],
                            acc.at[pl.ds(r0, rpt_last)])
            pltpu.sync_copy(table_h.at[pl.ds(r0, rpt_last)],
                            tbl.at[pl.ds(r0, rpt_last)])

        plsc.subcore_barrier()

        mask_hi = jnp.int32(-65536)

        def scale(rows_ref, sbuf_ref, j):
            def grp(g, carry):
                wv = w_v[j, pl.ds(g * _LANES, _LANES)]
                for l in range(_LANES):
                    ws = _splat_lane(wv, l)
                    e = g * _LANES + l
                    for q in range(2):
                        v = rows_ref[e, pl.ds(q * 32, 32)]
                        wrd = plsc.bitcast(v, jnp.int32)
                        lo = plsc.bitcast(lax.shift_left(wrd, 16), F32)
                        hi = plsc.bitcast(lax.bitwise_and(wrd, mask_hi), F32)
                        sbuf_ref[e, pl.ds(q * 32, _LANES)] = lo * ws
                        sbuf_ref[e, pl.ds(q * 32 + 16, _LANES)] = hi * ws
                return carry
            lax.fori_loop(0, _CHUNK // _LANES, grp, 0)

        for b in range(nring):
            pltpu.async_copy(tbl.at[src_v.at[b]], rows.at[b], gsems.at[b])

        def step(j4, carry):
            for b in range(nring):
                j = j4 * nring + b
                sb = b % nsring
                pltpu.make_async_copy(
                    tbl.at[src_v.at[j]], rows.at[b], gsems.at[b]).wait()

                @pl.when(j >= nsring)
                def _():
                    pltpu.make_async_copy(
                        sbuf.at[sb], acc.at[dst_v.at[j]], ssems.at[sb]).wait()

                scale(rows.at[b], sbuf.at[sb], j)

                @pl.when(j + nring < nch)
                def _():
                    pltpu.async_copy(tbl.at[src_v.at[j + nring]],
                                     rows.at[b], gsems.at[b])

                pltpu.async_copy(sbuf.at[sb], acc.at[dst_v.at[j]],
                                 ssems.at[sb], add=True)
            return carry

        lax.fori_loop(0, nch // nring, step, 0)
        for b in range(nsring):
            pltpu.make_async_copy(
                sbuf.at[b], acc.at[dst_v.at[nch - nsring + b]],
                ssems.at[b]).wait()
        plsc.subcore_barrier()

        @pl.when(s < _NSUB - 1)
        def _():
            pltpu.sync_copy(acc.at[pl.ds(r0, rpt)],
                            out_h.at[c, pl.ds(r0, rpt)])

        @pl.when(s == _NSUB - 1)
        def _():
            pltpu.sync_copy(acc.at[pl.ds(r0, rpt_last)],
                            out_h.at[c, pl.ds(r0, rpt_last)])

    call = pl.kernel(
        body,
        out_type=jax.ShapeDtypeStruct((_NCORES, n_nodes, feat), F32),
        mesh=mesh,
        scratch_types=[
            pltpu.VMEM((nch, _CHUNK), jnp.int32),
            pltpu.VMEM((nch, _CHUNK), jnp.int32),
            pltpu.VMEM((nch, _CHUNK), F32),
            pltpu.VMEM((nring, _CHUNK, feat), jnp.bfloat16),
            pltpu.VMEM((nsring, _CHUNK, feat), F32),
            pltpu.VMEM_SHARED((n_nodes, feat), jnp.bfloat16),
            pltpu.VMEM_SHARED((n_nodes, feat), F32),
            pltpu.SemaphoreType.DMA((nring,)),
            pltpu.SemaphoreType.DMA((nsring,)),
        ],
        compiler_params=pltpu.CompilerParams(use_tc_tiling_on_sc=False,
                                             needs_layout_passes=False),
    )
    return call(table, src_t, dst_t, w_t, zeros)


def _proj(x2d, w, block_rows=2000):
    rows, _ = x2d.shape
    din, dout = w.shape

    def body(x_ref, w_ref, o_ref):
        o_ref[...] = jnp.dot(
            x_ref[...], w_ref[...],
            preferred_element_type=F32).astype(jnp.bfloat16)

    return pl.pallas_call(
        body,
        grid=(rows // block_rows,),
        in_specs=[
            pl.BlockSpec((block_rows, din), lambda i: (i, 0)),
            pl.BlockSpec((din, dout), lambda i: (0, 0)),
        ],
        out_specs=pl.BlockSpec((block_rows, dout), lambda i: (i, 0)),
        out_shape=jax.ShapeDtypeStruct((rows, dout), jnp.bfloat16),
    )(x2d, w)


def _gru(p, h, bg, wr, wz, wn, ur, uz, un, brc, bzc, bin_, bhn, w_next=None,
         block_rows=2000):
    n, d = h.shape
    with_next = w_next is not None

    def body(*refs):
        if with_next:
            (p_ref, h_ref, bg_r, wr_r, wz_r, wn_r, ur_r, uz_r, un_r,
             brc_r, bzc_r, bin_r, bhn_r, wnx_r, ho_r, hw_r) = refs
        else:
            (p_ref, h_ref, bg_r, wr_r, wz_r, wn_r, ur_r, uz_r, un_r,
             brc_r, bzc_r, bin_r, bhn_r, ho_r) = refs
        dot = lambda a, m: jnp.dot(a, m[...], preferred_element_type=F32)
        xg = jnp.maximum(p_ref[0] + p_ref[1] + bg_r[...], 0.0)
        hh = h_ref[...]
        r = jax.nn.sigmoid(dot(xg, wr_r) + dot(hh, ur_r) + brc_r[...])
        z = jax.nn.sigmoid(dot(xg, wz_r) + dot(hh, uz_r) + bzc_r[...])
        nn = jnp.tanh(dot(xg, wn_r) + bin_r[...] + r * (dot(hh, un_r) + bhn_r[...]))
        h_new = (1.0 - z) * nn + z * hh
        ho_r[...] = h_new
        if with_next:
            hw_r[...] = dot(h_new, wnx_r).astype(jnp.bfloat16)

    mat = lambda: pl.BlockSpec((d, d), lambda i: (0, 0))
    vec = lambda: pl.BlockSpec((1, d), lambda i: (0, 0))
    in_specs = [
        pl.BlockSpec((2, block_rows, d), lambda i: (0, i, 0)),
        pl.BlockSpec((block_rows, d), lambda i: (i, 0)),
        vec(), mat(), mat(), mat(), mat(), mat(), mat(),
        vec(), vec(), vec(), vec(),
    ]
    args = [p, h, bg, wr, wz, wn, ur, uz, un, brc, bzc, bin_, bhn]
    out_spec = pl.BlockSpec((block_rows, d), lambda i: (i, 0))
    out_shape = jax.ShapeDtypeStruct((n, d), F32)
    if with_next:
        in_specs.append(mat())
        args.append(w_next)
        out_specs = [out_spec, out_spec]
        out_shapes = [out_shape, jax.ShapeDtypeStruct((n, d), jnp.bfloat16)]
    else:
        out_specs = out_spec
        out_shapes = out_shape
    return pl.pallas_call(
        body,
        grid=(n // block_rows,),
        in_specs=in_specs,
        out_specs=out_specs,
        out_shape=out_shapes,
    )(*args)


def _head(h1, ow1, ob1, ow2, ob2, block_rows=2000):
    n, d = h1.shape
    dhid = ow1.shape[1]
    steps = n // block_rows

    def body(h_ref, w1_ref, b1_ref, w2_ref, b2_ref, o_ref, acc_ref):
        i = pl.program_id(0)

        @pl.when(i == 0)
        def _():
            acc_ref[...] = jnp.zeros_like(acc_ref)

        acc_ref[...] += jnp.sum(h_ref[...], axis=0, keepdims=True)

        @pl.when(i == steps - 1)
        def _():
            g = acc_ref[...] * (1.0 / n)
            hid = jnp.maximum(
                jnp.dot(g, w1_ref[...], preferred_element_type=F32)
                + b1_ref[...], 0.0)
            o_ref[...] = jax.nn.sigmoid(
                jnp.dot(hid, w2_ref[...], preferred_element_type=F32)
                + b2_ref[...])

    return pl.pallas_call(
        body,
        grid=(steps,),
        in_specs=[
            pl.BlockSpec((block_rows, d), lambda i: (i, 0)),
            pl.BlockSpec((d, dhid), lambda i: (0, 0)),
            pl.BlockSpec((1, dhid), lambda i: (0, 0)),
            pl.BlockSpec((dhid, 1), lambda i: (0, 0)),
            pl.BlockSpec((1, 1), lambda i: (0, 0)),
        ],
        out_specs=pl.BlockSpec((1, 1), lambda i: (0, 0)),
        out_shape=jax.ShapeDtypeStruct((1, 1), F32),
        scratch_shapes=[pltpu.VMEM((1, d), F32)],
    )(h1, ow1, ob1.reshape(1, dhid), ow2, ob2.reshape(1, 1))


def _split_gru_params(wih, whh, bih, bhh, d):
    wr, wz, wn = wih[:, :d], wih[:, d:2 * d], wih[:, 2 * d:]
    ur, uz, un = whh[:, :d], whh[:, d:2 * d], whh[:, 2 * d:]
    brc = (bih[:d] + bhh[:d]).reshape(1, d)
    bzc = (bih[d:2 * d] + bhh[d:2 * d]).reshape(1, d)
    bin_ = bih[2 * d:].reshape(1, d)
    bhn = bhh[2 * d:].reshape(1, d)
    return wr, wz, wn, ur, uz, un, brc, bzc, bin_, bhn


def kernel(x_seq, edge_index, edge_weight, W0, b0, Wih0, Whh0, bih0, bhh0,
           W1, b1, Wih1, Whh1, bih1, bhh1, outW1, outb1, outW2, outb2):
    T, N, din = x_seq.shape
    dh = W0.shape[1]
    dout = W1.shape[1]
    E = edge_weight.shape[0]
    assert dh == 64 and dout == 64, "SC path specialized to 64-wide features"
    assert N % _NSUB == 0

    ept = -(-E // _NW)
    nch = -(-ept // _CHUNK)
    nch = -(-nch // 4) * 4
    pad_e = _NW * nch * _CHUNK - E
    src_t = jnp.pad(edge_index[0], (0, pad_e)).reshape(_NW, nch, _CHUNK)
    dst_t = jnp.pad(edge_index[1], (0, pad_e)).reshape(_NW, nch, _CHUNK)
    w_t = jnp.pad(edge_weight, (0, pad_e)).reshape(_NW, nch, _CHUNK)
    zeros = jnp.zeros((N, dh), F32)

    g0 = _split_gru_params(Wih0, Whh0, bih0, bhh0, dh)
    g1 = _split_gru_params(Wih1, Whh1, bih1, bhh1, dout)
    b0r = b0.reshape(1, dh)
    b1r = b1.reshape(1, dout)

    q = np.arange(dh) // 32
    r = np.arange(dh) % 32
    sigma = q * 32 + np.where(r < 16, 2 * r, 2 * (r - 16) + 1)
    tau = np.argsort(sigma)
    W0p = W0[:, tau]
    W1p = W1[:, tau]

    xw = _proj(x_seq.reshape(T * N, din), W0p).reshape(T, N, dh)

    h0 = jnp.zeros((N, dh), F32)
    h1 = jnp.zeros((N, dout), F32)
    p0s = [_spmm(xw[t], src_t, dst_t, w_t, zeros, N, nch) for t in range(T)]
    for t in range(T):
        p0 = p0s[t]
        h0, hw = _gru(p0, h0, b0r, *g0, w_next=W1p)
        p1 = _spmm(hw, src_t, dst_t, w_t, zeros, N, nch)
        h1 = _gru(p1, h1, b1r, *g1)
    return _head(h1, outW1, outb1, outW2, outb2)


# speedup vs baseline: 2.4373x; 2.2436x over previous
import functools

import jax
import jax.numpy as jnp
import numpy as np
from jax import lax
from jax.experimental import pallas as pl
from jax.experimental.pallas import tpu as pltpu
from jax.experimental.pallas import tpu_sc as plsc

F32 = jnp.float32
_LANES = 16
_CHUNK = 128
_NCORES = 2
_NSUB = 16
_NW = _NCORES * _NSUB


def _splat_lane(vec16, lane):
    idx = jnp.full((_LANES, 1), lane, jnp.int32)
    return lax.gather(
        vec16,
        idx,
        dimension_numbers=lax.GatherDimensionNumbers(
            offset_dims=(), collapsed_slice_dims=(0,), start_index_map=(0,)),
        slice_sizes=(1,),
        mode=lax.GatherScatterMode.PROMISE_IN_BOUNDS)


@functools.partial(jax.jit, static_argnums=(5, 6))
def _spmm(table, src_t, dst_t, w_t, zeros, n_nodes, nch):
    feat = 64
    nring = 4
    nsring = 2
    rpt = (-(-n_nodes // _NSUB) + 7) // 8 * 8
    rpt_last = n_nodes - (_NSUB - 1) * rpt
    assert rpt_last > 0 and rpt_last % 8 == 0
    assert nch % nring == 0
    mesh = plsc.VectorSubcoreMesh(core_axis_name="c", subcore_axis_name="s")

    def body(table_h, src_h, dst_h, w_h, zeros_h, out_h,
             src_v, dst_v, w_v, rows, sbuf, tbl, acc, gsems, ssems):
        c = lax.axis_index("c")
        s = lax.axis_index("s")
        wid = c * _NSUB + s
        pltpu.sync_copy(src_h.at[wid], src_v)
        pltpu.sync_copy(dst_h.at[wid], dst_v)
        pltpu.sync_copy(w_h.at[wid], w_v)
        r0 = s * rpt

        @pl.when(s < _NSUB - 1)
        def _():
            pltpu.sync_copy(zeros_h.at[pl.ds(r0, rpt)],
                            acc.at[pl.ds(r0, rpt)])
            pltpu.sync_copy(table_h.at[pl.ds(r0, rpt)],
                            tbl.at[pl.ds(r0, rpt)])

        @pl.when(s == _NSUB - 1)
        def _():
            pltpu.sync_copy(zeros_h.at[pl.ds(r0, rpt_last)],
                            acc.at[pl.ds(r0, rpt_last)])
            pltpu.sync_copy(table_h.at[pl.ds(r0, rpt_last)],
                            tbl.at[pl.ds(r0, rpt_last)])

        plsc.subcore_barrier()

        def scale(rows_ref, sbuf_ref, j):
            for g in range(_CHUNK // _LANES):
                wv = w_v[j, pl.ds(g * _LANES, _LANES)]
                for l in range(_LANES):
                    ws = _splat_lane(wv, l)
                    wsb = plsc.pack(ws, ws, format=plsc.PackFormat.INTERLEAVED)
                    e = g * _LANES + l
                    for q in range(2):
                        cs = pl.ds(q * 32, 32)
                        sbuf_ref[e, cs] = rows_ref[e, cs] * wsb

        for b in range(nring):
            pltpu.async_copy(tbl.at[src_v.at[b]], rows.at[b], gsems.at[b])

        def step(j4, carry):
            for b in range(nring):
                j = j4 * nring + b
                sb = b % nsring
                pltpu.make_async_copy(
                    tbl.at[src_v.at[j]], rows.at[b], gsems.at[b]).wait()

                @pl.when(j >= nsring)
                def _():
                    pltpu.make_async_copy(
                        sbuf.at[sb], acc.at[dst_v.at[j]], ssems.at[sb]).wait()

                scale(rows.at[b], sbuf.at[sb], j)

                @pl.when(j + nring < nch)
                def _():
                    pltpu.async_copy(tbl.at[src_v.at[j + nring]],
                                     rows.at[b], gsems.at[b])

                pltpu.async_copy(sbuf.at[sb], acc.at[dst_v.at[j]],
                                 ssems.at[sb], add=True)
            return carry

        lax.fori_loop(0, nch // nring, step, 0)
        for b in range(nsring):
            pltpu.make_async_copy(
                sbuf.at[b], acc.at[dst_v.at[nch - nsring + b]],
                ssems.at[b]).wait()
        plsc.subcore_barrier()

        @pl.when(s < _NSUB - 1)
        def _():
            pltpu.sync_copy(acc.at[pl.ds(r0, rpt)],
                            out_h.at[c, pl.ds(r0, rpt)])

        @pl.when(s == _NSUB - 1)
        def _():
            pltpu.sync_copy(acc.at[pl.ds(r0, rpt_last)],
                            out_h.at[c, pl.ds(r0, rpt_last)])

    call = pl.kernel(
        body,
        out_type=jax.ShapeDtypeStruct((_NCORES, n_nodes, feat), jnp.bfloat16),
        mesh=mesh,
        scratch_types=[
            pltpu.VMEM((nch, _CHUNK), jnp.int32),
            pltpu.VMEM((nch, _CHUNK), jnp.int32),
            pltpu.VMEM((nch, _CHUNK), F32),
            pltpu.VMEM((nring, _CHUNK, feat), jnp.bfloat16),
            pltpu.VMEM((nsring, _CHUNK, feat), jnp.bfloat16),
            pltpu.VMEM_SHARED((n_nodes, feat), jnp.bfloat16),
            pltpu.VMEM_SHARED((n_nodes, feat), jnp.bfloat16),
            pltpu.SemaphoreType.DMA((nring,)),
            pltpu.SemaphoreType.DMA((nsring,)),
        ],
        compiler_params=pltpu.CompilerParams(use_tc_tiling_on_sc=False,
                                             needs_layout_passes=False),
    )
    return call(table, src_t, dst_t, w_t, zeros)


def _proj(x2d, w, block_rows=2000):
    rows, _ = x2d.shape
    din, dout = w.shape

    def body(x_ref, w_ref, o_ref):
        o_ref[...] = jnp.dot(
            x_ref[...], w_ref[...],
            preferred_element_type=F32).astype(jnp.bfloat16)

    return pl.pallas_call(
        body,
        grid=(rows // block_rows,),
        in_specs=[
            pl.BlockSpec((block_rows, din), lambda i: (i, 0)),
            pl.BlockSpec((din, dout), lambda i: (0, 0)),
        ],
        out_specs=pl.BlockSpec((block_rows, dout), lambda i: (i, 0)),
        out_shape=jax.ShapeDtypeStruct((rows, dout), jnp.bfloat16),
    )(x2d, w)


def _gru(p, h, bg, wr, wz, wn, ur, uz, un, brc, bzc, bin_, bhn, w_next=None,
         block_rows=2000):
    n, d = h.shape
    with_next = w_next is not None

    def body(*refs):
        if with_next:
            (p_ref, h_ref, bg_r, wr_r, wz_r, wn_r, ur_r, uz_r, un_r,
             brc_r, bzc_r, bin_r, bhn_r, wnx_r, ho_r, hw_r) = refs
        else:
            (p_ref, h_ref, bg_r, wr_r, wz_r, wn_r, ur_r, uz_r, un_r,
             brc_r, bzc_r, bin_r, bhn_r, ho_r) = refs
        dot = lambda a, m: jnp.dot(a, m[...], preferred_element_type=F32)
        agg = p_ref[0].astype(F32) + p_ref[1].astype(F32)
        xg = jnp.maximum(agg + bg_r[...], 0.0)
        hh = h_ref[...]
        r = jax.nn.sigmoid(dot(xg, wr_r) + dot(hh, ur_r) + brc_r[...])
        z = jax.nn.sigmoid(dot(xg, wz_r) + dot(hh, uz_r) + bzc_r[...])
        nn = jnp.tanh(dot(xg, wn_r) + bin_r[...] + r * (dot(hh, un_r) + bhn_r[...]))
        h_new = (1.0 - z) * nn + z * hh
        ho_r[...] = h_new
        if with_next:
            hw_r[...] = dot(h_new, wnx_r).astype(jnp.bfloat16)

    mat = lambda: pl.BlockSpec((d, d), lambda i: (0, 0))
    vec = lambda: pl.BlockSpec((1, d), lambda i: (0, 0))
    in_specs = [
        pl.BlockSpec((2, block_rows, d), lambda i: (0, i, 0)),
        pl.BlockSpec((block_rows, d), lambda i: (i, 0)),
        vec(), mat(), mat(), mat(), mat(), mat(), mat(),
        vec(), vec(), vec(), vec(),
    ]
    args = [p, h, bg, wr, wz, wn, ur, uz, un, brc, bzc, bin_, bhn]
    out_spec = pl.BlockSpec((block_rows, d), lambda i: (i, 0))
    out_shape = jax.ShapeDtypeStruct((n, d), F32)
    if with_next:
        in_specs.append(mat())
        args.append(w_next)
        out_specs = [out_spec, out_spec]
        out_shapes = [out_shape, jax.ShapeDtypeStruct((n, d), jnp.bfloat16)]
    else:
        out_specs = out_spec
        out_shapes = out_shape
    return pl.pallas_call(
        body,
        grid=(n // block_rows,),
        in_specs=in_specs,
        out_specs=out_specs,
        out_shape=out_shapes,
    )(*args)


def _head(h1, ow1, ob1, ow2, ob2, block_rows=2000):
    n, d = h1.shape
    dhid = ow1.shape[1]
    steps = n // block_rows

    def body(h_ref, w1_ref, b1_ref, w2_ref, b2_ref, o_ref, acc_ref):
        i = pl.program_id(0)

        @pl.when(i == 0)
        def _():
            acc_ref[...] = jnp.zeros_like(acc_ref)

        acc_ref[...] += jnp.sum(h_ref[...], axis=0, keepdims=True)

        @pl.when(i == steps - 1)
        def _():
            g = acc_ref[...] * (1.0 / n)
            hid = jnp.maximum(
                jnp.dot(g, w1_ref[...], preferred_element_type=F32)
                + b1_ref[...], 0.0)
            o_ref[...] = jax.nn.sigmoid(
                jnp.dot(hid, w2_ref[...], preferred_element_type=F32)
                + b2_ref[...])

    return pl.pallas_call(
        body,
        grid=(steps,),
        in_specs=[
            pl.BlockSpec((block_rows, d), lambda i: (i, 0)),
            pl.BlockSpec((d, dhid), lambda i: (0, 0)),
            pl.BlockSpec((1, dhid), lambda i: (0, 0)),
            pl.BlockSpec((dhid, 1), lambda i: (0, 0)),
            pl.BlockSpec((1, 1), lambda i: (0, 0)),
        ],
        out_specs=pl.BlockSpec((1, 1), lambda i: (0, 0)),
        out_shape=jax.ShapeDtypeStruct((1, 1), F32),
        scratch_shapes=[pltpu.VMEM((1, d), F32)],
    )(h1, ow1, ob1.reshape(1, dhid), ow2, ob2.reshape(1, 1))


def _split_gru_params(wih, whh, bih, bhh, d):
    wr, wz, wn = wih[:, :d], wih[:, d:2 * d], wih[:, 2 * d:]
    ur, uz, un = whh[:, :d], whh[:, d:2 * d], whh[:, 2 * d:]
    brc = (bih[:d] + bhh[:d]).reshape(1, d)
    bzc = (bih[d:2 * d] + bhh[d:2 * d]).reshape(1, d)
    bin_ = bih[2 * d:].reshape(1, d)
    bhn = bhh[2 * d:].reshape(1, d)
    return wr, wz, wn, ur, uz, un, brc, bzc, bin_, bhn


def kernel(x_seq, edge_index, edge_weight, W0, b0, Wih0, Whh0, bih0, bhh0,
           W1, b1, Wih1, Whh1, bih1, bhh1, outW1, outb1, outW2, outb2):
    T, N, din = x_seq.shape
    dh = W0.shape[1]
    dout = W1.shape[1]
    E = edge_weight.shape[0]
    assert dh == 64 and dout == 64, "SC path specialized to 64-wide features"
    assert N % _NSUB == 0

    ept = -(-E // _NW)
    nch = -(-ept // _CHUNK)
    nch = -(-nch // 4) * 4
    pad_e = _NW * nch * _CHUNK - E
    src_t = jnp.pad(edge_index[0], (0, pad_e)).reshape(_NW, nch, _CHUNK)
    dst_t = jnp.pad(edge_index[1], (0, pad_e)).reshape(_NW, nch, _CHUNK)
    w_t = jnp.pad(edge_weight, (0, pad_e)).reshape(_NW, nch, _CHUNK)
    zeros = jnp.zeros((N, dh), jnp.bfloat16)

    g0 = _split_gru_params(Wih0, Whh0, bih0, bhh0, dh)
    g1 = _split_gru_params(Wih1, Whh1, bih1, bhh1, dout)
    b0r = b0.reshape(1, dh)
    b1r = b1.reshape(1, dout)

    xw = _proj(x_seq.reshape(T * N, din), W0).reshape(T, N, dh)

    h0 = jnp.zeros((N, dh), F32)
    h1 = jnp.zeros((N, dout), F32)
    p0s = [_spmm(xw[t], src_t, dst_t, w_t, zeros, N, nch) for t in range(T)]
    for t in range(T):
        p0 = p0s[t]
        h0, hw = _gru(p0, h0, b0r, *g0, w_next=W1)
        p1 = _spmm(hw, src_t, dst_t, w_t, zeros, N, nch)
        h1 = _gru(p1, h1, b1r, *g1)
    return _head(h1, outW1, outb1, outW2, outb2)
